# baseline (speedup 1.0000x reference)
import functools

import jax
import jax.numpy as jnp
from jax import lax
from jax.experimental import pallas as pl
from jax.experimental.pallas import tpu as pltpu
from jax.experimental.pallas import tpu_sc as plsc

NUM_SC = 2
NUM_TILES = 16
NUM_WORKERS = NUM_SC * NUM_TILES
DEG_W = 8
LANES = 16


def _sc_mesh():
    return plsc.VectorSubcoreMesh(core_axis_name="c", subcore_axis_name="s")


_SC_PARAMS = pltpu.CompilerParams(use_tc_tiling_on_sc=False)


def _sc_degree(dst3, n_pad):
    _, nchunk, c = dst3.shape
    rpt = n_pad // NUM_TILES

    @functools.partial(
        pl.kernel,
        out_type=jax.ShapeDtypeStruct((NUM_SC, n_pad, DEG_W), jnp.float32),
        mesh=_sc_mesh(),
        compiler_params=_SC_PARAMS,
        scratch_types=[
            pltpu.VMEM((nchunk, c), jnp.int32),
            pltpu.VMEM((c, DEG_W), jnp.float32),
            pltpu.VMEM_SHARED((n_pad, DEG_W), jnp.float32),
        ],
    )
    def k(dst_hbm, ones_hbm, zeros_hbm, out_hbm, dsti_v, ones_v, acc_sh):
        cid = lax.axis_index("c")
        sid = lax.axis_index("s")
        wid = sid * NUM_SC + cid
        pltpu.sync_copy(dst_hbm.at[wid], dsti_v)
        pltpu.sync_copy(ones_hbm, ones_v)
        pltpu.sync_copy(zeros_hbm, acc_sh.at[pl.ds(sid * rpt, rpt)])
        plsc.subcore_barrier()

        def body(j, carry):
            pltpu.sync_copy(ones_v, acc_sh.at[dsti_v.at[j]], add=True)
            return carry

        lax.fori_loop(0, nchunk, body, 0)
        plsc.subcore_barrier()
        pltpu.sync_copy(
            acc_sh.at[pl.ds(sid * rpt, rpt)],
            out_hbm.at[cid, pl.ds(sid * rpt, rpt)],
        )

    ones = jnp.ones((c, DEG_W), jnp.float32)
    zeros = jnp.zeros((rpt, DEG_W), jnp.float32)
    return k(dst3, ones, zeros)


def _sc_aggregate(h2, src2, dst3, n_pad, d, planes):
    nplanes = len(planes)
    out_w = d * nplanes
    _, nchunk, c = dst3.shape
    epw = src2.shape[1]
    rpt = n_pad // NUM_TILES
    need_t = any(m != 1 or a != 0 for m, a, _ in planes)

    @functools.partial(
        pl.kernel,
        out_type=jax.ShapeDtypeStruct((NUM_SC, n_pad, out_w), jnp.float32),
        mesh=_sc_mesh(),
        compiler_params=_SC_PARAMS,
        scratch_types=[
            pltpu.VMEM((epw,), jnp.int32),
            pltpu.VMEM((epw if need_t else LANES,), jnp.int32),
            pltpu.VMEM((nchunk, c), jnp.int32),
            pltpu.VMEM((4, c, d), jnp.float32),
            pltpu.VMEM_SHARED((n_pad, d), jnp.float32),
            [pltpu.SemaphoreType.DMA] * 4,
            [pltpu.SemaphoreType.DMA] * 4,
        ],
    )
    def k(h_hbm, src_hbm, dst_hbm, zeros_hbm, out_hbm,
          srci_v, srct_v, dsti_v, rows_v, acc_sh, gsem, ssem):
        cid = lax.axis_index("c")
        sid = lax.axis_index("s")
        wid = sid * NUM_SC + cid
        pltpu.sync_copy(src_hbm.at[wid], srci_v)
        pltpu.sync_copy(dst_hbm.at[wid], dsti_v)

        for mul, add, out_col in planes:
            if mul == 1 and add == 0:
                idx_v = srci_v
            else:
                def tbody(i, carry, mul=mul, add=add):
                    v = srci_v[pl.ds(i * LANES, LANES)]
                    srct_v[pl.ds(i * LANES, LANES)] = v * mul + add
                    return carry

                lax.fori_loop(0, epw // LANES, tbody, 0)
                idx_v = srct_v

            def gath(jj, b, idx_v=idx_v):
                pltpu.async_copy(h_hbm.at[idx_v.at[pl.ds(jj * c, c)]],
                                 rows_v.at[b], gsem[b])

            def gath_wait(jj, b, idx_v=idx_v):
                pltpu.make_async_copy(h_hbm.at[idx_v.at[pl.ds(jj * c, c)]],
                                      rows_v.at[b], gsem[b]).wait()

            def scat(jj, b):
                pltpu.async_copy(rows_v.at[b], acc_sh.at[dsti_v.at[jj]],
                                 ssem[b], add=True)

            def scat_wait(jj, b):
                pltpu.make_async_copy(rows_v.at[b], acc_sh.at[dsti_v.at[jj]],
                                      ssem[b]).wait()

            pltpu.sync_copy(zeros_hbm, acc_sh.at[pl.ds(sid * rpt, rpt)])
            gath(0, 0)
            gath(1, 1)
            plsc.subcore_barrier()

            for b in range(4):
                gath_wait(b, b % 4)
                if b >= 1:
                    scat_wait(b - 1, (b - 1) % 4)
                scat(b, b % 4)
                gath(b + 2, (b + 2) % 4)

            def body(jh, carry):
                for b in range(4):
                    jj = 4 * jh + b
                    gath_wait(jj, b)
                    scat_wait(jj - 1, (b + 3) % 4)
                    scat(jj, b)

                    @pl.when(jj + 2 < nchunk)
                    def _():
                        gath(jj + 2, (b + 2) % 4)
                return carry

            lax.fori_loop(1, nchunk // 4, body, 0)
            scat_wait(nchunk - 1, (nchunk - 1) % 4)
            plsc.subcore_barrier()
            pltpu.sync_copy(
                acc_sh.at[pl.ds(sid * rpt, rpt)],
                out_hbm.at[cid, pl.ds(sid * rpt, rpt), pl.ds(out_col, d)],
            )

    zeros = jnp.zeros((rpt, d), jnp.float32)
    return k(h2, src2, dst3, zeros)


def _tc_layer1(x, w1, degp):
    n, d_in = x.shape
    d_h = w1.shape[1]

    def body(x_ref, w_ref, degp_ref, hp_ref, dis_ref):
        deg = degp_ref[0, :n, 0:1] + degp_ref[1, :n, 0:1] + 1.0
        dis = lax.rsqrt(deg)
        h = jnp.dot(x_ref[...], w_ref[...],
                    preferred_element_type=jnp.float32,
                    precision=lax.Precision.HIGHEST)
        hp_ref[...] = h * dis
        dis_ref[...] = dis

    return pl.pallas_call(
        body,
        out_shape=[
            jax.ShapeDtypeStruct((n, d_h), jnp.float32),
            jax.ShapeDtypeStruct((n, 1), jnp.float32),
        ],
    )(x, w1, degp)


def _tc_layer2(p, hp, dis, b1, w2):
    n, d_h = hp.shape
    d_o = w2.shape[1]

    def body(p_ref, hp_ref, dis_ref, b1_ref, w2_ref, gp_ref):
        dis = dis_ref[...]
        z = jnp.maximum(dis * (p_ref[0, :n] + p_ref[1, :n] + hp_ref[...])
                        + b1_ref[...], 0.0)
        g = jnp.dot(z, w2_ref[...],
                    preferred_element_type=jnp.float32,
                    precision=lax.Precision.HIGHEST)
        gp_ref[...] = g * dis

    return pl.pallas_call(
        body,
        out_shape=jax.ShapeDtypeStruct((n, d_o), jnp.float32),
    )(p, hp, dis, b1, w2)


def _tc_final(q, gp, dis, b2):
    n, d_o = gp.shape

    def body(q_ref, gp_ref, dis_ref, b2_ref, out_ref):
        s = q_ref[0, :n] + q_ref[1, :n] + gp_ref[...]
        out_ref[...] = dis_ref[...] * s + b2_ref[...]

    return pl.pallas_call(
        body,
        out_shape=jax.ShapeDtypeStruct((n, d_o), jnp.float32),
    )(q, gp, dis, b2)


def kernel(x, edge_index, W1, b1, W2, b2):
    n, _ = x.shape
    e = edge_index.shape[1]

    n_pad = ((n + 1 + NUM_TILES * 8 - 1) // (NUM_TILES * 8)) * (NUM_TILES * 8)

    c = 128
    nchunk = -(-e // (NUM_WORKERS * c))
    nchunk = ((nchunk + 3) // 4) * 4
    epw = nchunk * c
    e_pad = NUM_WORKERS * epw
    ei = edge_index.astype(jnp.int32)
    src = jnp.concatenate([ei[0], jnp.zeros((e_pad - e,), jnp.int32)])
    dst = jnp.concatenate([ei[1], jnp.full((e_pad - e,), n, jnp.int32)])
    src2 = src.reshape(NUM_WORKERS, epw)
    dst3 = dst.reshape(NUM_WORKERS, nchunk, c)

    degp = _sc_degree(dst3, n_pad)

    hp, dis = _tc_layer1(x, W1, degp)
    dh2 = hp.shape[1] // 2
    hp2 = hp.reshape(2 * n, dh2)
    p = _sc_aggregate(hp2, src2, dst3, n_pad, dh2, [(2, 0, 0), (2, 1, dh2)])
    gp = _tc_layer2(p, hp, dis, b1.reshape(1, -1), W2)
    q = _sc_aggregate(gp, src2, dst3, n_pad, gp.shape[1], [(1, 0, 0)])
    return _tc_final(q, gp, dis, b2.reshape(1, -1))


# speedup vs baseline: 1.0044x; 1.0044x over previous
import functools

import jax
import jax.numpy as jnp
from jax import lax
from jax.experimental import pallas as pl
from jax.experimental.pallas import tpu as pltpu
from jax.experimental.pallas import tpu_sc as plsc

NUM_SC = 2
NUM_TILES = 16
NUM_WORKERS = NUM_SC * NUM_TILES
DEG_W = 8
LANES = 16


def _sc_mesh():
    return plsc.VectorSubcoreMesh(core_axis_name="c", subcore_axis_name="s")


_SC_PARAMS = pltpu.CompilerParams(use_tc_tiling_on_sc=False)


def _sc_degree(dst3, n_pad):
    _, nchunk, c = dst3.shape
    rpt = n_pad // NUM_TILES

    @functools.partial(
        pl.kernel,
        out_type=jax.ShapeDtypeStruct((NUM_SC, n_pad, DEG_W), jnp.float32),
        mesh=_sc_mesh(),
        compiler_params=_SC_PARAMS,
        scratch_types=[
            pltpu.VMEM((nchunk, c), jnp.int32),
            pltpu.VMEM((c, DEG_W), jnp.float32),
            pltpu.VMEM_SHARED((n_pad, DEG_W), jnp.float32),
        ],
    )
    def k(dst_hbm, ones_hbm, zeros_hbm, out_hbm, dsti_v, ones_v, acc_sh):
        cid = lax.axis_index("c")
        sid = lax.axis_index("s")
        wid = sid * NUM_SC + cid
        pltpu.sync_copy(dst_hbm.at[wid], dsti_v)
        pltpu.sync_copy(ones_hbm, ones_v)
        pltpu.sync_copy(zeros_hbm, acc_sh.at[pl.ds(sid * rpt, rpt)])
        plsc.subcore_barrier()

        def body(j, carry):
            pltpu.sync_copy(ones_v, acc_sh.at[dsti_v.at[j]], add=True)
            return carry

        lax.fori_loop(0, nchunk, body, 0)
        plsc.subcore_barrier()
        pltpu.sync_copy(
            acc_sh.at[pl.ds(sid * rpt, rpt)],
            out_hbm.at[cid, pl.ds(sid * rpt, rpt)],
        )

    ones = jnp.ones((c, DEG_W), jnp.float32)
    zeros = jnp.zeros((rpt, DEG_W), jnp.float32)
    return k(dst3, ones, zeros)


def _sc_aggregate(h2, src3, dst3, n_pad, d, planes):
    nplanes = len(planes)
    out_w = d * nplanes
    _, nchunk, c = dst3.shape
    rpt = n_pad // NUM_TILES
    need_t = any(m != 1 or a != 0 for m, a, _ in planes)

    @functools.partial(
        pl.kernel,
        out_type=jax.ShapeDtypeStruct((NUM_SC, n_pad, out_w), jnp.float32),
        mesh=_sc_mesh(),
        compiler_params=_SC_PARAMS,
        scratch_types=[
            pltpu.VMEM((nchunk, c), jnp.int32),
            pltpu.VMEM((nchunk if need_t else 1, c), jnp.int32),
            pltpu.VMEM((nchunk, c), jnp.int32),
            pltpu.VMEM((4, c, d), jnp.float32),
            pltpu.VMEM_SHARED((n_pad, d), jnp.float32),
            [pltpu.SemaphoreType.DMA] * 4,
            [pltpu.SemaphoreType.DMA] * 4,
        ],
    )
    def k(h_hbm, src_hbm, dst_hbm, zeros_hbm, out_hbm,
          srci_v, srct_v, dsti_v, rows_v, acc_sh, gsem, ssem):
        cid = lax.axis_index("c")
        sid = lax.axis_index("s")
        wid = sid * NUM_SC + cid
        pltpu.sync_copy(src_hbm.at[wid], srci_v)
        pltpu.sync_copy(dst_hbm.at[wid], dsti_v)

        for mul, add, out_col in planes:
            if mul == 1 and add == 0:
                idx_v = srci_v
            else:
                def tbody(r, carry, mul=mul, add=add):
                    for kk in range(c // LANES):
                        v = srci_v[r, pl.ds(kk * LANES, LANES)]
                        srct_v[r, pl.ds(kk * LANES, LANES)] = v * mul + add
                    return carry

                lax.fori_loop(0, nchunk, tbody, 0)
                idx_v = srct_v

            def gath(jj, b, idx_v=idx_v):
                pltpu.async_copy(h_hbm.at[idx_v.at[jj]],
                                 rows_v.at[b], gsem[b])

            def gath_wait(jj, b, idx_v=idx_v):
                pltpu.make_async_copy(h_hbm.at[idx_v.at[jj]],
                                      rows_v.at[b], gsem[b]).wait()

            def scat(jj, b):
                pltpu.async_copy(rows_v.at[b], acc_sh.at[dsti_v.at[jj]],
                                 ssem[b], add=True)

            def scat_wait(jj, b):
                pltpu.make_async_copy(rows_v.at[b], acc_sh.at[dsti_v.at[jj]],
                                      ssem[b]).wait()

            pltpu.sync_copy(zeros_hbm, acc_sh.at[pl.ds(sid * rpt, rpt)])
            gath(0, 0)
            gath(1, 1)
            plsc.subcore_barrier()

            for b in range(4):
                gath_wait(b, b % 4)
                if b >= 1:
                    scat_wait(b - 1, (b - 1) % 4)
                scat(b, b % 4)
                gath(b + 2, (b + 2) % 4)

            def body(jh, carry):
                for b in range(4):
                    jj = 4 * jh + b
                    gath_wait(jj, b)
                    scat_wait(jj - 1, (b + 3) % 4)
                    scat(jj, b)

                    @pl.when(jj + 2 < nchunk)
                    def _():
                        gath(jj + 2, (b + 2) % 4)
                return carry

            lax.fori_loop(1, nchunk // 4, body, 0)
            scat_wait(nchunk - 1, (nchunk - 1) % 4)
            plsc.subcore_barrier()
            pltpu.sync_copy(
                acc_sh.at[pl.ds(sid * rpt, rpt)],
                out_hbm.at[cid, pl.ds(sid * rpt, rpt), pl.ds(out_col, d)],
            )

    zeros = jnp.zeros((rpt, d), jnp.float32)
    return k(h2, src3, dst3, zeros)


def _tc_layer1(x, w1, degp):
    n, d_in = x.shape
    d_h = w1.shape[1]

    def body(x_ref, w_ref, degp_ref, hp_ref, dis_ref):
        deg = degp_ref[0, :n, 0:1] + degp_ref[1, :n, 0:1] + 1.0
        dis = lax.rsqrt(deg)
        h = jnp.dot(x_ref[...], w_ref[...],
                    preferred_element_type=jnp.float32,
                    precision=lax.Precision.HIGHEST)
        hp_ref[...] = h * dis
        dis_ref[...] = dis

    return pl.pallas_call(
        body,
        out_shape=[
            jax.ShapeDtypeStruct((n, d_h), jnp.float32),
            jax.ShapeDtypeStruct((n, 1), jnp.float32),
        ],
    )(x, w1, degp)


def _tc_layer2(p, hp, dis, b1, w2):
    n, d_h = hp.shape
    d_o = w2.shape[1]

    def body(p_ref, hp_ref, dis_ref, b1_ref, w2_ref, gp_ref):
        dis = dis_ref[...]
        z = jnp.maximum(dis * (p_ref[0, :n] + p_ref[1, :n] + hp_ref[...])
                        + b1_ref[...], 0.0)
        g = jnp.dot(z, w2_ref[...],
                    preferred_element_type=jnp.float32,
                    precision=lax.Precision.HIGHEST)
        gp_ref[...] = g * dis

    return pl.pallas_call(
        body,
        out_shape=jax.ShapeDtypeStruct((n, d_o), jnp.float32),
    )(p, hp, dis, b1, w2)


def _tc_final(q, gp, dis, b2):
    n, d_o = gp.shape

    def body(q_ref, gp_ref, dis_ref, b2_ref, out_ref):
        s = q_ref[0, :n] + q_ref[1, :n] + gp_ref[...]
        out_ref[...] = dis_ref[...] * s + b2_ref[...]

    return pl.pallas_call(
        body,
        out_shape=jax.ShapeDtypeStruct((n, d_o), jnp.float32),
    )(q, gp, dis, b2)


def kernel(x, edge_index, W1, b1, W2, b2):
    n, _ = x.shape
    e = edge_index.shape[1]

    n_pad = ((n + 1 + NUM_TILES * 8 - 1) // (NUM_TILES * 8)) * (NUM_TILES * 8)

    c = 128
    nchunk = -(-e // (NUM_WORKERS * c))
    nchunk = ((nchunk + 3) // 4) * 4
    epw = nchunk * c
    e_pad = NUM_WORKERS * epw
    ei = edge_index.astype(jnp.int32)
    src = jnp.concatenate([ei[0], jnp.zeros((e_pad - e,), jnp.int32)])
    dst = jnp.concatenate([ei[1], jnp.full((e_pad - e,), n, jnp.int32)])
    src3 = src.reshape(NUM_WORKERS, nchunk, c)
    dst3 = dst.reshape(NUM_WORKERS, nchunk, c)

    degp = _sc_degree(dst3, n_pad)

    hp, dis = _tc_layer1(x, W1, degp)
    dh2 = hp.shape[1] // 2
    hp2 = hp.reshape(2 * n, dh2)
    p = _sc_aggregate(hp2, src3, dst3, n_pad, dh2, [(2, 0, 0), (2, 1, dh2)])
    gp = _tc_layer2(p, hp, dis, b1.reshape(1, -1), W2)
    q = _sc_aggregate(gp, src3, dst3, n_pad, gp.shape[1], [(1, 0, 0)])
    return _tc_final(q, gp, dis, b2.reshape(1, -1))
